# baseline (speedup 1.0000x reference)
import functools

import jax
import jax.numpy as jnp
from jax import lax
from jax.experimental import pallas as pl
from jax.experimental.pallas import tpu as pltpu
from jax.experimental.pallas import tpu_sc as plsc

B, H, W = 16, 512, 512
N = H * W
L = 16
NC, NS = 2, 16
NW = NC * NS
NBINS = 1024
CHUNK = 16384
NCHUNK = N // CHUNK
NGRP = NBINS // L
UNROLL = 2
POS_T = 0.1

_mesh = plsc.VectorSubcoreMesh(
    core_axis_name="c", subcore_axis_name="s", num_cores=NC, num_subcores=NS
)


def _suffix_incl(x, carry):
    sfx = jnp.flip(jnp.cumsum(jnp.flip(x, 0)), 0) + carry
    new_carry = carry + jnp.broadcast_to(jnp.sum(x), (L,))
    return sfx, new_carry


@functools.partial(
    pl.kernel,
    out_type=jax.ShapeDtypeStruct((NW, L), jnp.float32),
    mesh=_mesh,
    compiler_params=pltpu.CompilerParams(needs_layout_passes=False),
    scratch_types=[
        pltpu.VMEM((CHUNK,), jnp.float32),
        pltpu.VMEM((CHUNK,), jnp.float32),
        pltpu.VMEM((CHUNK,), jnp.float32),
        pltpu.VMEM((CHUNK,), jnp.float32),
        pltpu.VMEM((CHUNK,), jnp.float32),
        pltpu.VMEM((CHUNK,), jnp.float32),
        pltpu.VMEM((L * NBINS,), jnp.float32),
        pltpu.VMEM((L,), jnp.float32),
        pltpu.SemaphoreType.DMA,
        pltpu.SemaphoreType.DMA,
    ],
)
def _sc_loss(gh, gah, pg, pga, mk, out,
             la_v, pa_v, ma_v, lb_v, pb_v, mb_v,
             hc_v, res_v, sem_a, sem_b):
    cid = lax.axis_index("c")
    sid = lax.axis_index("s")
    wid = sid * NC + cid
    lane_iota = lax.iota(jnp.int32, L)
    lane_f = lane_iota.astype(jnp.float32)
    lane_off = lane_iota * NBINS
    zeros = jnp.zeros((L,), jnp.float32)
    ones = jnp.ones((L,), jnp.float32)

    def run(lbl_hbm, p_hbm, img):
        base = img * N

        @plsc.parallel_loop(0, (L * NBINS) // (UNROLL * L))
        def _(i):
            for j in range(UNROLL):
                hc_v[pl.ds(i * (UNROLL * L) + j * L, L)] = zeros

        def start(off, l_v, p_v, m_v, sem):
            pltpu.async_copy(lbl_hbm.at[pl.ds(off, CHUNK)], l_v, sem)
            pltpu.async_copy(p_hbm.at[pl.ds(off, CHUNK)], p_v, sem)
            pltpu.async_copy(mk.at[pl.ds(off, CHUNK)], m_v, sem)

        def wait3(l_v, p_v, m_v, sem):
            src = lbl_hbm.at[pl.ds(0, CHUNK)]
            pltpu.make_async_copy(src, l_v, sem).wait()
            pltpu.make_async_copy(src, p_v, sem).wait()
            pltpu.make_async_copy(src, m_v, sem).wait()

        def process(l_v, p_v, m_v, tots):
            def inner(i, accs):
                ts, ns = accs
                ts_o, ns_o = [], []
                for j in range(UNROLL):
                    o = i * (UNROLL * L) + j * L
                    lb = l_v[pl.ds(o, L)]
                    pr = p_v[pl.ds(o, L)]
                    mm = m_v[pl.ds(o, L)]
                    d = pr - lb
                    v = d * d * mm
                    neg = lb < POS_T
                    bn = jnp.minimum((v * NBINS).astype(jnp.int32), NBINS - 1)
                    plsc.addupdate_scatter(hc_v, [lane_off + bn], ones, mask=neg)
                    ts_o.append(ts[j] + v)
                    ns_o.append(ns[j] + jnp.where(neg, v, zeros))
                return (tuple(ts_o), tuple(ns_o))

            return plsc.parallel_loop(
                0, CHUNK // (UNROLL * L), carry=tots
            )(inner)

        start(base, la_v, pa_v, ma_v, sem_a)

        def pair_body(pi, tots):
            off = base + pi * (2 * CHUNK)
            wait3(la_v, pa_v, ma_v, sem_a)
            start(off + CHUNK, lb_v, pb_v, mb_v, sem_b)
            tots = process(la_v, pa_v, ma_v, tots)
            wait3(lb_v, pb_v, mb_v, sem_b)

            @pl.when(pi < NCHUNK // 2 - 1)
            def _():
                start(off + 2 * CHUNK, la_v, pa_v, ma_v, sem_a)

            return process(lb_v, pb_v, mb_v, tots)

        tots0 = (tuple(zeros for _ in range(UNROLL)),
                 tuple(zeros for _ in range(UNROLL)))
        ts_f, ns_f = lax.fori_loop(0, NCHUNK // 2, pair_body, tots0)
        tot_p = zeros
        neg_p = zeros
        for a, b in zip(ts_f, ns_f):
            tot_p = tot_p + a
            neg_p = neg_p + b
        tot = jnp.broadcast_to(jnp.sum(tot_p), (L,))
        neg_sum = jnp.broadcast_to(jnp.sum(neg_p), (L,))

        def cnt_body(g, acc):
            c = zeros
            for l in range(L):
                c = c + hc_v[pl.ds(l * NBINS + g * L, L)]
            return acc + c

        pre_cc = lax.fori_loop(0, NGRP, cnt_body, zeros)
        neg_n = jnp.broadcast_to(jnp.sum(pre_cc), (L,))
        pos_n = float(N) - neg_n
        k_v = jnp.clip(3.0 * pos_n, 1.0, float(N))
        k500_v = jnp.full((L,), 500.0, jnp.float32)

        def walk_body(j, carry):
            cc, cs, acc_k, acc_500 = carry
            g = (NGRP - 1) - j
            c = zeros
            for l in range(L):
                c = c + hc_v[pl.ds(l * NBINS + g * L, L)]
            centers = (jnp.float32(g * L) + lane_f + 0.5) * (1.0 / NBINS)
            s = c * centers
            C, cc = _suffix_incl(c, cc)
            S, cs = _suffix_incl(s, cs)

            def pick(kk):
                m = jnp.logical_and(C >= kk, (C - c) < kk)
                return jnp.where(m, S - (C - kk) * centers, zeros)

            return (cc, cs, acc_k + pick(k_v), acc_500 + pick(k500_v))

        _, _, acc_k, acc_500 = lax.fori_loop(
            0, NGRP, walk_body, (zeros, zeros, zeros, zeros)
        )
        topk_mean = jnp.broadcast_to(jnp.sum(acc_k), (L,)) / k_v
        top500_mean = jnp.broadcast_to(jnp.sum(acc_500), (L,)) / k500_v

        posi = (tot - neg_sum) / jnp.maximum(pos_n, ones)
        nega_mean = neg_sum / jnp.maximum(neg_n, ones)
        nega = jnp.where(neg_n < 3.0 * pos_n, nega_mean, topk_mean)
        res = jnp.where(pos_n > 0.0, posi + nega, top500_mean)

        res_v[...] = res
        pltpu.sync_copy(res_v, out.at[wid])

    @pl.when(wid < B)
    def _():
        run(gh, pg, wid)

    @pl.when(wid >= B)
    def _():
        run(gah, pga, wid - B)


def kernel(gh_label, gah_label, p_gh, p_gah, mask):
    flat = lambda x: x.reshape(B * N)
    out = _sc_loss(flat(gh_label), flat(gah_label), flat(p_gh), flat(p_gah),
                   flat(mask))
    return jnp.sum(out[:, 0]) / B


# speedup vs baseline: 1.0266x; 1.0266x over previous
import functools

import jax
import jax.numpy as jnp
from jax import lax
from jax.experimental import pallas as pl
from jax.experimental.pallas import tpu as pltpu
from jax.experimental.pallas import tpu_sc as plsc

B, H, W = 16, 512, 512
N = H * W
L = 16
NC, NS = 2, 16
NW = NC * NS
NBINS = 1024
CHUNK = 16384
NCHUNK = N // CHUNK
NGRP = NBINS // L
UNROLL = 4
POS_T = 0.1

_mesh = plsc.VectorSubcoreMesh(
    core_axis_name="c", subcore_axis_name="s", num_cores=NC, num_subcores=NS
)


def _suffix_incl(x, carry):
    sfx = jnp.flip(jnp.cumsum(jnp.flip(x, 0)), 0) + carry
    new_carry = carry + jnp.broadcast_to(jnp.sum(x), (L,))
    return sfx, new_carry


@functools.partial(
    pl.kernel,
    out_type=jax.ShapeDtypeStruct((NW, L), jnp.float32),
    mesh=_mesh,
    compiler_params=pltpu.CompilerParams(needs_layout_passes=False),
    scratch_types=[
        pltpu.VMEM((CHUNK,), jnp.float32),
        pltpu.VMEM((CHUNK,), jnp.float32),
        pltpu.VMEM((CHUNK,), jnp.float32),
        pltpu.VMEM((CHUNK,), jnp.float32),
        pltpu.VMEM((CHUNK,), jnp.float32),
        pltpu.VMEM((CHUNK,), jnp.float32),
        pltpu.VMEM((L * NBINS,), jnp.float32),
        pltpu.VMEM((L,), jnp.float32),
        pltpu.SemaphoreType.DMA,
        pltpu.SemaphoreType.DMA,
    ],
)
def _sc_loss(gh, gah, pg, pga, mk, out,
             la_v, pa_v, ma_v, lb_v, pb_v, mb_v,
             hc_v, res_v, sem_a, sem_b):
    cid = lax.axis_index("c")
    sid = lax.axis_index("s")
    wid = sid * NC + cid
    lane_iota = lax.iota(jnp.int32, L)
    lane_f = lane_iota.astype(jnp.float32)
    lane_off = lane_iota * NBINS
    zeros = jnp.zeros((L,), jnp.float32)
    ones = jnp.ones((L,), jnp.float32)

    def run(lbl_hbm, p_hbm, img):
        base = img * N

        @plsc.parallel_loop(0, (L * NBINS) // (UNROLL * L))
        def _(i):
            for j in range(UNROLL):
                hc_v[pl.ds(i * (UNROLL * L) + j * L, L)] = zeros

        def start(off, l_v, p_v, m_v, sem):
            pltpu.async_copy(lbl_hbm.at[pl.ds(off, CHUNK)], l_v, sem)
            pltpu.async_copy(p_hbm.at[pl.ds(off, CHUNK)], p_v, sem)
            pltpu.async_copy(mk.at[pl.ds(off, CHUNK)], m_v, sem)

        def wait3(l_v, p_v, m_v, sem):
            src = lbl_hbm.at[pl.ds(0, CHUNK)]
            pltpu.make_async_copy(src, l_v, sem).wait()
            pltpu.make_async_copy(src, p_v, sem).wait()
            pltpu.make_async_copy(src, m_v, sem).wait()

        def process(l_v, p_v, m_v, tots):
            def inner(i, accs):
                ts, ns = accs
                ts_o, ns_o = [], []
                for j in range(UNROLL):
                    o = i * (UNROLL * L) + j * L
                    lb = l_v[pl.ds(o, L)]
                    pr = p_v[pl.ds(o, L)]
                    mm = m_v[pl.ds(o, L)]
                    d = pr - lb
                    v = d * d * mm
                    neg = lb < POS_T
                    bn = jnp.minimum((v * NBINS).astype(jnp.int32), NBINS - 1)
                    plsc.addupdate_scatter(hc_v, [lane_off + bn], ones, mask=neg)
                    ts_o.append(ts[j] + v)
                    ns_o.append(ns[j] + jnp.where(neg, v, zeros))
                return (tuple(ts_o), tuple(ns_o))

            return plsc.parallel_loop(
                0, CHUNK // (UNROLL * L), carry=tots
            )(inner)

        start(base, la_v, pa_v, ma_v, sem_a)

        def pair_body(pi, tots):
            off = base + pi * (2 * CHUNK)
            wait3(la_v, pa_v, ma_v, sem_a)
            start(off + CHUNK, lb_v, pb_v, mb_v, sem_b)
            tots = process(la_v, pa_v, ma_v, tots)
            wait3(lb_v, pb_v, mb_v, sem_b)

            @pl.when(pi < NCHUNK // 2 - 1)
            def _():
                start(off + 2 * CHUNK, la_v, pa_v, ma_v, sem_a)

            return process(lb_v, pb_v, mb_v, tots)

        tots0 = (tuple(zeros for _ in range(UNROLL)),
                 tuple(zeros for _ in range(UNROLL)))
        ts_f, ns_f = lax.fori_loop(0, NCHUNK // 2, pair_body, tots0)
        tot_p = zeros
        neg_p = zeros
        for a, b in zip(ts_f, ns_f):
            tot_p = tot_p + a
            neg_p = neg_p + b
        tot = jnp.broadcast_to(jnp.sum(tot_p), (L,))
        neg_sum = jnp.broadcast_to(jnp.sum(neg_p), (L,))

        def cnt_body(g, acc):
            c = zeros
            for l in range(L):
                c = c + hc_v[pl.ds(l * NBINS + g * L, L)]
            return acc + c

        pre_cc = lax.fori_loop(0, NGRP, cnt_body, zeros)
        neg_n = jnp.broadcast_to(jnp.sum(pre_cc), (L,))
        pos_n = float(N) - neg_n
        k_v = jnp.clip(3.0 * pos_n, 1.0, float(N))
        k500_v = jnp.full((L,), 500.0, jnp.float32)

        def walk_body(j, carry):
            cc, cs, acc_k, acc_500 = carry
            g = (NGRP - 1) - j
            c = zeros
            for l in range(L):
                c = c + hc_v[pl.ds(l * NBINS + g * L, L)]
            centers = (jnp.float32(g * L) + lane_f + 0.5) * (1.0 / NBINS)
            s = c * centers
            C, cc = _suffix_incl(c, cc)
            S, cs = _suffix_incl(s, cs)

            def pick(kk):
                m = jnp.logical_and(C >= kk, (C - c) < kk)
                return jnp.where(m, S - (C - kk) * centers, zeros)

            return (cc, cs, acc_k + pick(k_v), acc_500 + pick(k500_v))

        _, _, acc_k, acc_500 = lax.fori_loop(
            0, NGRP, walk_body, (zeros, zeros, zeros, zeros)
        )
        topk_mean = jnp.broadcast_to(jnp.sum(acc_k), (L,)) / k_v
        top500_mean = jnp.broadcast_to(jnp.sum(acc_500), (L,)) / k500_v

        posi = (tot - neg_sum) / jnp.maximum(pos_n, ones)
        nega_mean = neg_sum / jnp.maximum(neg_n, ones)
        nega = jnp.where(neg_n < 3.0 * pos_n, nega_mean, topk_mean)
        res = jnp.where(pos_n > 0.0, posi + nega, top500_mean)

        res_v[...] = res
        pltpu.sync_copy(res_v, out.at[wid])

    @pl.when(wid < B)
    def _():
        run(gh, pg, wid)

    @pl.when(wid >= B)
    def _():
        run(gah, pga, wid - B)


def kernel(gh_label, gah_label, p_gh, p_gah, mask):
    flat = lambda x: x.reshape(B * N)
    out = _sc_loss(flat(gh_label), flat(gah_label), flat(p_gh), flat(p_gah),
                   flat(mask))
    return jnp.sum(out[:, 0]) / B


# speedup vs baseline: 1.0502x; 1.0230x over previous
import functools

import jax
import jax.numpy as jnp
from jax import lax
from jax.experimental import pallas as pl
from jax.experimental.pallas import tpu as pltpu
from jax.experimental.pallas import tpu_sc as plsc

B, H, W = 16, 512, 512
N = H * W
HALF = N // 2
L = 16
NC, NS = 2, 16
NW = NC * NS
NBINS = 1024
CHUNK = 8192
NCHUNK = HALF // CHUNK
NGRP = NBINS // L
UNROLL = 4
EXROW = 2 * NBINS + 128
POS_T = 0.1

_mesh = plsc.VectorSubcoreMesh(
    core_axis_name="c", subcore_axis_name="s", num_cores=NC, num_subcores=NS
)


def _suffix_incl(x, carry):
    sfx = jnp.flip(jnp.cumsum(jnp.flip(x, 0)), 0) + carry
    new_carry = carry + jnp.broadcast_to(jnp.sum(x), (L,))
    return sfx, new_carry


@functools.partial(
    pl.kernel,
    out_type=jax.ShapeDtypeStruct((NW, L), jnp.float32),
    mesh=_mesh,
    compiler_params=pltpu.CompilerParams(needs_layout_passes=False),
    scratch_types=[
        pltpu.VMEM((CHUNK,), jnp.float32),
        pltpu.VMEM((CHUNK,), jnp.float32),
        pltpu.VMEM((CHUNK,), jnp.float32),
        pltpu.VMEM((CHUNK,), jnp.float32),
        pltpu.VMEM((CHUNK,), jnp.float32),
        pltpu.VMEM((CHUNK,), jnp.float32),
        pltpu.VMEM((CHUNK,), jnp.float32),
        pltpu.VMEM((CHUNK,), jnp.float32),
        pltpu.VMEM((CHUNK,), jnp.float32),
        pltpu.VMEM((CHUNK,), jnp.float32),
        pltpu.VMEM((L * NBINS,), jnp.float32),
        pltpu.VMEM((L * NBINS,), jnp.float32),
        pltpu.VMEM((EXROW,), jnp.float32),
        pltpu.VMEM((EXROW,), jnp.float32),
        pltpu.VMEM((EXROW,), jnp.float32),
        pltpu.VMEM((L,), jnp.float32),
        pltpu.VMEM_SHARED((NS, EXROW), jnp.float32),
        pltpu.SemaphoreType.DMA,
        pltpu.SemaphoreType.DMA,
    ],
)
def _sc_loss(gh, gah, pg, pga, mk, out,
             glA, gaA, pgA, paA, mA, glB, gaB, pgB, paB, mB,
             hcg_v, hca_v, mc_v, mh0_v, mh1_v, res_v,
             hist_sh, sem_a, sem_b):
    cid = lax.axis_index("c")
    sid = lax.axis_index("s")
    img = cid * 8 + sid // 2
    half = sid % 2
    base = img * N + half * HALF
    pair0 = (sid // 2) * 2
    lane_iota = lax.iota(jnp.int32, L)
    lane_f = lane_iota.astype(jnp.float32)
    lane_off = lane_iota * NBINS
    zeros = jnp.zeros((L,), jnp.float32)
    ones = jnp.ones((L,), jnp.float32)

    @plsc.parallel_loop(0, (L * NBINS) // (UNROLL * L))
    def _(i):
        for j in range(UNROLL):
            o = i * (UNROLL * L) + j * L
            hcg_v[pl.ds(o, L)] = zeros
            hca_v[pl.ds(o, L)] = zeros

    def start(off, bufs, sem):
        for src, dst in zip((gh, gah, pg, pga, mk), bufs):
            pltpu.async_copy(src.at[pl.ds(off, CHUNK)], dst, sem)

    def wait5(bufs, sem):
        src = gh.at[pl.ds(0, CHUNK)]
        for dst in bufs:
            pltpu.make_async_copy(src, dst, sem).wait()

    bufsA = (glA, gaA, pgA, paA, mA)
    bufsB = (glB, gaB, pgB, paB, mB)

    def process(bufs, tots):
        gl_v, ga_v, pg_v, pa_v, m_v = bufs

        def inner(i, accs):
            tsg, nsg, tsa, nsa = accs
            tsg, nsg, tsa, nsa = list(tsg), list(nsg), list(tsa), list(nsa)
            for j in range(UNROLL):
                o = i * (UNROLL * L) + j * L
                lg = gl_v[pl.ds(o, L)]
                la = ga_v[pl.ds(o, L)]
                prg = pg_v[pl.ds(o, L)]
                pra = pa_v[pl.ds(o, L)]
                mm = m_v[pl.ds(o, L)]
                dg = prg - lg
                vg = dg * dg * mm
                da = pra - la
                va = da * da * mm
                negg = lg < POS_T
                nega = la < POS_T
                bng = jnp.minimum((vg * NBINS).astype(jnp.int32), NBINS - 1)
                bna = jnp.minimum((va * NBINS).astype(jnp.int32), NBINS - 1)
                plsc.addupdate_scatter(hcg_v, [lane_off + bng], ones, mask=negg)
                plsc.addupdate_scatter(hca_v, [lane_off + bna], ones, mask=nega)
                tsg[j] = tsg[j] + vg
                nsg[j] = nsg[j] + jnp.where(negg, vg, zeros)
                tsa[j] = tsa[j] + va
                nsa[j] = nsa[j] + jnp.where(nega, va, zeros)
            return (tuple(tsg), tuple(nsg), tuple(tsa), tuple(nsa))

        return plsc.parallel_loop(0, CHUNK // (UNROLL * L), carry=tots)(inner)

    start(base, bufsA, sem_a)

    def pair_body(pi, tots):
        off = base + pi * (2 * CHUNK)
        wait5(bufsA, sem_a)
        start(off + CHUNK, bufsB, sem_b)
        tots = process(bufsA, tots)
        wait5(bufsB, sem_b)

        @pl.when(pi < NCHUNK // 2 - 1)
        def _():
            start(off + 2 * CHUNK, bufsA, sem_a)

        return process(bufsB, tots)

    z4 = tuple(zeros for _ in range(UNROLL))
    tsg, nsg, tsa, nsa = lax.fori_loop(0, NCHUNK // 2, pair_body,
                                       (z4, z4, z4, z4))

    def _fold(parts):
        r = parts[0]
        for a in parts[1:]:
            r = r + a
        return jnp.broadcast_to(jnp.sum(r), (L,))

    SOFF = 2 * NBINS
    mc_v[pl.ds(SOFF + 0 * L, L)] = _fold(tsg)
    mc_v[pl.ds(SOFF + 1 * L, L)] = _fold(nsg)
    mc_v[pl.ds(SOFF + 2 * L, L)] = _fold(tsa)
    mc_v[pl.ds(SOFF + 3 * L, L)] = _fold(nsa)

    @plsc.parallel_loop(0, NGRP)
    def _(g):
        cg = zeros
        ca = zeros
        for l in range(L):
            o = l * NBINS + g * L
            cg = cg + hcg_v[pl.ds(o, L)]
            ca = ca + hca_v[pl.ds(o, L)]
        mc_v[pl.ds(g * L, L)] = cg
        mc_v[pl.ds(NBINS + g * L, L)] = ca

    pltpu.sync_copy(mc_v, hist_sh.at[sid])
    plsc.subcore_barrier()
    pltpu.sync_copy(hist_sh.at[pair0], mh0_v)
    pltpu.sync_copy(hist_sh.at[pair0 + 1], mh1_v)

    def finish(loss):
        hoff = loss * NBINS
        so = 2 * NBINS + 2 * loss * L
        tot = mh0_v[pl.ds(so, L)] + mh1_v[pl.ds(so, L)]
        neg_sum = mh0_v[pl.ds(so + L, L)] + mh1_v[pl.ds(so + L, L)]

        def cnt_body(g, acc):
            o = hoff + g * L
            return acc + mh0_v[pl.ds(o, L)] + mh1_v[pl.ds(o, L)]

        pre_cc = lax.fori_loop(0, NGRP, cnt_body, zeros)
        neg_n = jnp.broadcast_to(jnp.sum(pre_cc), (L,))
        pos_n = float(N) - neg_n
        k_v = jnp.clip(3.0 * pos_n, 1.0, float(N))
        k500_v = jnp.full((L,), 500.0, jnp.float32)

        def walk_body(j, carry):
            cc, cs, acc_k, acc_500 = carry
            g = (NGRP - 1) - j
            o = hoff + g * L
            c = mh0_v[pl.ds(o, L)] + mh1_v[pl.ds(o, L)]
            centers = (jnp.float32(g * L) + lane_f + 0.5) * (1.0 / NBINS)
            s = c * centers
            C, cc = _suffix_incl(c, cc)
            S, cs = _suffix_incl(s, cs)

            def pick(kk):
                m = jnp.logical_and(C >= kk, (C - c) < kk)
                return jnp.where(m, S - (C - kk) * centers, zeros)

            return (cc, cs, acc_k + pick(k_v), acc_500 + pick(k500_v))

        _, _, acc_k, acc_500 = lax.fori_loop(
            0, NGRP, walk_body, (zeros, zeros, zeros, zeros)
        )
        topk_mean = jnp.broadcast_to(jnp.sum(acc_k), (L,)) / k_v
        top500_mean = jnp.broadcast_to(jnp.sum(acc_500), (L,)) / k500_v

        posi = (tot - neg_sum) / jnp.maximum(pos_n, ones)
        nega_mean = neg_sum / jnp.maximum(neg_n, ones)
        nega = jnp.where(neg_n < 3.0 * pos_n, nega_mean, topk_mean)
        res = jnp.where(pos_n > 0.0, posi + nega, top500_mean)

        res_v[...] = res
        pltpu.sync_copy(res_v, out.at[loss * B + img])

    @pl.when(half == 0)
    def _():
        finish(0)

    @pl.when(half == 1)
    def _():
        finish(1)


def kernel(gh_label, gah_label, p_gh, p_gah, mask):
    flat = lambda x: x.reshape(B * N)
    out = _sc_loss(flat(gh_label), flat(gah_label), flat(p_gh), flat(p_gah),
                   flat(mask))
    return jnp.sum(out[:, 0]) / B
